# baseline (speedup 1.0000x reference)
import functools

import jax
import jax.numpy as jnp
from jax import lax
from jax.experimental import pallas as pl
from jax.experimental.pallas import tpu as pltpu
from jax.experimental.pallas import tpu_sc as plsc

E = 8
K = 2
H = 1024
F = 768
M = 2048
BLK = 128
NBLK = 39
NPAD = NBLK * BLK
NC, NS = 2, 16
NW = NC * NS
TPW = M // NW


def _router_body(x_ref, rw_ref, pos0_ref, pos1_ref, w0_ref, w1_ref, be_ref):
    x = x_ref[...]
    rw = rw_ref[...]
    lt = lax.dot_general(rw, x, (((0,), (1,)), ((), ())),
                         preferred_element_type=jnp.float32)

    m1 = jnp.max(lt, axis=0, keepdims=True)
    sel1 = lt == m1
    masked = jnp.where(sel1, -1e30, lt)
    m2 = jnp.max(masked, axis=0, keepdims=True)
    sel2 = masked == m2

    r = jnp.exp(m2 - m1)
    w0 = 1.0 / (1.0 + r)
    w0_ref[...] = w0.reshape(M)
    w1_ref[...] = (1.0 - w0).reshape(M)

    memb = jnp.where(sel1 | sel2, 1.0, 0.0)
    zc = jnp.zeros((E, M), jnp.float32)
    c = memb
    for k in range(11):
        s = 1 << k
        c = c + jnp.concatenate([zc[:, :s], c[:, : M - s]], axis=1)
    cex = c - memb

    counts = c[:, M - 1 : M]
    cnt_i = counts.astype(jnp.int32)
    padded_i = ((cnt_i + (BLK - 1)) >> 7) << 7
    padded_f = padded_i.astype(jnp.float32)
    zo = jnp.zeros((E, 1), jnp.float32)
    o = padded_f
    for k in range(3):
        s = 1 << k
        o = o + jnp.concatenate([zo[:s, :] * 0.0, o[: E - s, :]], axis=0)
    off = o - padded_f

    dest = off + cex
    pos0 = jnp.sum(jnp.where(sel1, dest, 0.0), axis=0)
    pos1 = jnp.sum(jnp.where(sel2, dest, 0.0), axis=0)
    pos0_ref[...] = pos0.astype(jnp.int32)
    pos1_ref[...] = pos1.astype(jnp.int32)

    ends = (off + padded_f) * (1.0 / BLK)
    bl = lax.broadcasted_iota(jnp.int32, (E, 128), 1).astype(jnp.float32)
    be = jnp.sum(jnp.where(ends <= bl, 1, 0), axis=0, keepdims=True)
    be_ref[...] = jnp.minimum(be, E - 1).astype(jnp.int32)


_router = pl.pallas_call(
    _router_body,
    out_shape=[
        jax.ShapeDtypeStruct((M,), jnp.int32),
        jax.ShapeDtypeStruct((M,), jnp.int32),
        jax.ShapeDtypeStruct((M,), jnp.float32),
        jax.ShapeDtypeStruct((M,), jnp.float32),
        jax.ShapeDtypeStruct((1, 128), jnp.int32),
    ],
    compiler_params=pltpu.CompilerParams(vmem_limit_bytes=100 * 1024 * 1024),
)


@functools.cache
def _sc_kernels():
    mesh = plsc.VectorSubcoreMesh(core_axis_name="c", subcore_axis_name="s",
                                  num_cores=NC, num_subcores=NS)

    @functools.partial(
        pl.kernel,
        out_type=jax.ShapeDtypeStruct((NPAD, H), jnp.float32),
        mesh=mesh,
        scratch_types=[
            pltpu.VMEM((TPW,), jnp.int32),
            pltpu.VMEM((TPW,), jnp.int32),
            pltpu.VMEM((TPW, H), jnp.float32),
            pltpu.SemaphoreType.DMA,
        ],
    )
    def dispatch(x_hbm, pos0_hbm, pos1_hbm, xs_hbm, idx0_v, idx1_v, rows_v, sem):
        wid = lax.axis_index("s") * NC + lax.axis_index("c")
        base = wid * TPW
        pltpu.sync_copy(pos0_hbm.at[pl.ds(base, TPW)], idx0_v)
        pltpu.sync_copy(pos1_hbm.at[pl.ds(base, TPW)], idx1_v)
        pltpu.sync_copy(x_hbm.at[pl.ds(base, TPW)], rows_v)
        pltpu.async_copy(rows_v, xs_hbm.at[idx0_v], sem).wait()
        pltpu.async_copy(rows_v, xs_hbm.at[idx1_v], sem).wait()

    C = TPW // 2

    @functools.partial(
        pl.kernel,
        out_type=jax.ShapeDtypeStruct((M, H), jnp.float32),
        mesh=mesh,
        scratch_types=[
            pltpu.VMEM((TPW,), jnp.int32),
            pltpu.VMEM((TPW,), jnp.int32),
            pltpu.VMEM((TPW,), jnp.float32),
            pltpu.VMEM((TPW,), jnp.float32),
            pltpu.VMEM((C, H), jnp.float32),
            pltpu.VMEM((C, H), jnp.float32),
            pltpu.SemaphoreType.DMA,
        ],
        compiler_params=pltpu.CompilerParams(needs_layout_passes=False),
    )
    def gather_combine(ys_hbm, pos0_hbm, pos1_hbm, w0_hbm, w1_hbm, out_hbm,
                       idx0_v, idx1_v, w0_v, w1_v, buf0, buf1, sem):
        wid = lax.axis_index("s") * NC + lax.axis_index("c")
        base = wid * TPW
        pltpu.sync_copy(pos0_hbm.at[pl.ds(base, TPW)], idx0_v)
        pltpu.sync_copy(pos1_hbm.at[pl.ds(base, TPW)], idx1_v)
        pltpu.sync_copy(w0_hbm.at[pl.ds(base, TPW)], w0_v)
        pltpu.sync_copy(w1_hbm.at[pl.ds(base, TPW)], w1_v)
        for h in range(TPW // C):
            pltpu.async_copy(ys_hbm.at[idx0_v.at[pl.ds(h * C, C)]], buf0, sem).wait()
            pltpu.async_copy(ys_hbm.at[idx1_v.at[pl.ds(h * C, C)]], buf1, sem).wait()

            def row_body(rr, _, h=h):
                t = h * C + rr
                bidx = jnp.zeros((16,), jnp.int32) + t
                w0b = plsc.load_gather(w0_v, [bidx])
                w1b = plsc.load_gather(w1_v, [bidx])

                def col_body(cc, _):
                    b0 = buf0[rr, pl.ds(cc * 16, 16)]
                    b1 = buf1[rr, pl.ds(cc * 16, 16)]
                    buf0[rr, pl.ds(cc * 16, 16)] = b0 * w0b + b1 * w1b
                    return 0

                lax.fori_loop(0, H // 16, col_body, 0, unroll=8)
                return 0

            lax.fori_loop(0, C, row_body, 0)
            pltpu.sync_copy(buf0, out_hbm.at[pl.ds(base + h * C, C)])

    return dispatch, gather_combine


def _gemm_body(be_ref, xs_ref, w1_ref, w3_ref, w2_ref, out_ref):
    xb = xs_ref[...]
    h = jnp.dot(xb, w1_ref[0], preferred_element_type=jnp.float32)
    u = jnp.dot(xb, w3_ref[0], preferred_element_type=jnp.float32)
    act = h * (1.0 / (1.0 + jnp.exp(-h))) * u
    out_ref[...] = jnp.dot(act, w2_ref[0], preferred_element_type=jnp.float32)


_gemm = pl.pallas_call(
    _gemm_body,
    grid_spec=pltpu.PrefetchScalarGridSpec(
        num_scalar_prefetch=1,
        grid=(NBLK,),
        in_specs=[
            pl.BlockSpec((BLK, H), lambda b, be: (b, 0)),
            pl.BlockSpec((1, H, F), lambda b, be: (be[0, b], 0, 0)),
            pl.BlockSpec((1, H, F), lambda b, be: (be[0, b], 0, 0)),
            pl.BlockSpec((1, F, H), lambda b, be: (be[0, b], 0, 0)),
        ],
        out_specs=pl.BlockSpec((BLK, H), lambda b, be: (b, 0)),
    ),
    out_shape=jax.ShapeDtypeStruct((NPAD, H), jnp.float32),
    compiler_params=pltpu.CompilerParams(vmem_limit_bytes=100 * 1024 * 1024),
)


def _combine_body(y0_ref, y1_ref, w0_ref, w1_ref, o_ref):
    w0 = w0_ref[...].reshape(BLK, 1)
    w1 = w1_ref[...].reshape(BLK, 1)
    o_ref[...] = y0_ref[...] * w0 + y1_ref[...] * w1


_combine = pl.pallas_call(
    _combine_body,
    grid=(M // BLK,),
    in_specs=[
        pl.BlockSpec((BLK, H), lambda b: (b, 0)),
        pl.BlockSpec((BLK, H), lambda b: (b, 0)),
        pl.BlockSpec((BLK,), lambda b: (b,)),
        pl.BlockSpec((BLK,), lambda b: (b,)),
    ],
    out_specs=pl.BlockSpec((BLK, H), lambda b: (b, 0)),
    out_shape=jax.ShapeDtypeStruct((M, H), jnp.float32),
)


def kernel(x, router_w, w1, w3, w2):
    bs, seqlen, dim = x.shape
    xt = x.reshape(M, H)
    pos0, pos1, wt0, wt1, be = _router(xt, router_w)
    dispatch, gather_combine = _sc_kernels()
    xs = dispatch(xt, pos0, pos1)
    ys = _gemm(be, xs, w1, w3, w2)
    out = gather_combine(ys, pos0, pos1, wt0, wt1)
    return out.reshape(bs, seqlen, dim)


# speedup vs baseline: 1.0703x; 1.0703x over previous
import functools

import jax
import jax.numpy as jnp
from jax import lax
from jax.experimental import pallas as pl
from jax.experimental.pallas import tpu as pltpu
from jax.experimental.pallas import tpu_sc as plsc

E = 8
K = 2
H = 1024
F = 768
M = 2048
BLK = 128
NBLK = 39
NPAD = NBLK * BLK
NC, NS = 2, 16
NW = NC * NS
TPW = M // NW


def _router_body(x_ref, rw_ref, pos0_ref, pos1_ref, w0_ref, w1_ref, be_ref):
    x = x_ref[...]
    rw = rw_ref[...]
    lt = lax.dot_general(rw, x, (((0,), (1,)), ((), ())),
                         preferred_element_type=jnp.float32)

    m1 = jnp.max(lt, axis=0, keepdims=True)
    sel1 = lt == m1
    masked = jnp.where(sel1, -1e30, lt)
    m2 = jnp.max(masked, axis=0, keepdims=True)
    sel2 = masked == m2

    r = jnp.exp(m2 - m1)
    w0 = 1.0 / (1.0 + r)
    w0_ref[...] = w0.reshape(M)
    w1_ref[...] = (1.0 - w0).reshape(M)

    memb = jnp.where(sel1 | sel2, 1.0, 0.0)
    zc = jnp.zeros((E, M), jnp.float32)
    c = memb
    for k in range(11):
        s = 1 << k
        c = c + jnp.concatenate([zc[:, :s], c[:, : M - s]], axis=1)
    cex = c - memb

    counts = c[:, M - 1 : M]
    cnt_i = counts.astype(jnp.int32)
    padded_i = ((cnt_i + (BLK - 1)) >> 7) << 7
    padded_f = padded_i.astype(jnp.float32)
    zo = jnp.zeros((E, 1), jnp.float32)
    o = padded_f
    for k in range(3):
        s = 1 << k
        o = o + jnp.concatenate([zo[:s, :] * 0.0, o[: E - s, :]], axis=0)
    off = o - padded_f

    dest = off + cex
    pos0 = jnp.sum(jnp.where(sel1, dest, 0.0), axis=0)
    pos1 = jnp.sum(jnp.where(sel2, dest, 0.0), axis=0)
    pos0_ref[...] = pos0.astype(jnp.int32)
    pos1_ref[...] = pos1.astype(jnp.int32)

    ends = (off + padded_f) * (1.0 / BLK)
    bl = lax.broadcasted_iota(jnp.int32, (E, 128), 1).astype(jnp.float32)
    be = jnp.sum(jnp.where(ends <= bl, 1, 0), axis=0, keepdims=True)
    be_ref[...] = jnp.minimum(be, E - 1).astype(jnp.int32)


_router = pl.pallas_call(
    _router_body,
    out_shape=[
        jax.ShapeDtypeStruct((M,), jnp.int32),
        jax.ShapeDtypeStruct((M,), jnp.int32),
        jax.ShapeDtypeStruct((M,), jnp.float32),
        jax.ShapeDtypeStruct((M,), jnp.float32),
        jax.ShapeDtypeStruct((1, 128), jnp.int32),
    ],
    compiler_params=pltpu.CompilerParams(vmem_limit_bytes=100 * 1024 * 1024),
)


@functools.cache
def _sc_kernels():
    mesh = plsc.VectorSubcoreMesh(core_axis_name="c", subcore_axis_name="s",
                                  num_cores=NC, num_subcores=NS)

    @functools.partial(
        pl.kernel,
        out_type=jax.ShapeDtypeStruct((NPAD, H), jnp.float32),
        mesh=mesh,
        scratch_types=[
            pltpu.VMEM((TPW,), jnp.int32),
            pltpu.VMEM((TPW,), jnp.int32),
            pltpu.VMEM((TPW, H), jnp.float32),
            pltpu.SemaphoreType.DMA,
        ],
    )
    def dispatch(x_hbm, pos0_hbm, pos1_hbm, xs_hbm, idx0_v, idx1_v, rows_v, sem):
        wid = lax.axis_index("s") * NC + lax.axis_index("c")
        base = wid * TPW
        pltpu.sync_copy(pos0_hbm.at[pl.ds(base, TPW)], idx0_v)
        pltpu.sync_copy(pos1_hbm.at[pl.ds(base, TPW)], idx1_v)
        pltpu.sync_copy(x_hbm.at[pl.ds(base, TPW)], rows_v)
        a = pltpu.async_copy(rows_v, xs_hbm.at[idx0_v], sem)
        b = pltpu.async_copy(rows_v, xs_hbm.at[idx1_v], sem)
        a.wait()
        b.wait()

    @functools.partial(
        pl.kernel,
        out_type=[
            jax.ShapeDtypeStruct((M, H), jnp.float32),
            jax.ShapeDtypeStruct((M, H), jnp.float32),
        ],
        mesh=mesh,
        scratch_types=[
            pltpu.VMEM((TPW,), jnp.int32),
            pltpu.VMEM((TPW,), jnp.int32),
            pltpu.VMEM((TPW // 2, H), jnp.float32),
            pltpu.VMEM((TPW // 2, H), jnp.float32),
            pltpu.SemaphoreType.DMA,
        ],
    )
    def gather(ys_hbm, pos0_hbm, pos1_hbm, y0_hbm, y1_hbm, idx0_v, idx1_v,
               rows_a, rows_b, sem):
        wid = lax.axis_index("s") * NC + lax.axis_index("c")
        base = wid * TPW
        Ch = TPW // 2
        pltpu.sync_copy(pos0_hbm.at[pl.ds(base, TPW)], idx0_v)
        pltpu.sync_copy(pos1_hbm.at[pl.ds(base, TPW)], idx1_v)
        g = pltpu.async_copy(ys_hbm.at[idx0_v.at[pl.ds(0, Ch)]], rows_a, sem)
        h = pltpu.async_copy(ys_hbm.at[idx0_v.at[pl.ds(Ch, Ch)]], rows_b, sem)
        g.wait()
        s0 = pltpu.async_copy(rows_a, y0_hbm.at[pl.ds(base, Ch)], sem)
        h.wait()
        s1 = pltpu.async_copy(rows_b, y0_hbm.at[pl.ds(base + Ch, Ch)], sem)
        s0.wait()
        g = pltpu.async_copy(ys_hbm.at[idx1_v.at[pl.ds(0, Ch)]], rows_a, sem)
        s1.wait()
        h = pltpu.async_copy(ys_hbm.at[idx1_v.at[pl.ds(Ch, Ch)]], rows_b, sem)
        g.wait()
        s0 = pltpu.async_copy(rows_a, y1_hbm.at[pl.ds(base, Ch)], sem)
        h.wait()
        s1 = pltpu.async_copy(rows_b, y1_hbm.at[pl.ds(base + Ch, Ch)], sem)
        s0.wait()
        s1.wait()

    return dispatch, gather


def _gemm_body(be_ref, xs_ref, w1_ref, w3_ref, w2_ref, out_ref):
    xb = xs_ref[...]
    h = jnp.dot(xb, w1_ref[0], preferred_element_type=jnp.float32)
    u = jnp.dot(xb, w3_ref[0], preferred_element_type=jnp.float32)
    act = h * (1.0 / (1.0 + jnp.exp(-h))) * u
    out_ref[...] = jnp.dot(act, w2_ref[0], preferred_element_type=jnp.float32)


_gemm = pl.pallas_call(
    _gemm_body,
    grid_spec=pltpu.PrefetchScalarGridSpec(
        num_scalar_prefetch=1,
        grid=(NBLK,),
        in_specs=[
            pl.BlockSpec((BLK, H), lambda b, be: (b, 0)),
            pl.BlockSpec((1, H, F), lambda b, be: (be[0, b], 0, 0)),
            pl.BlockSpec((1, H, F), lambda b, be: (be[0, b], 0, 0)),
            pl.BlockSpec((1, F, H), lambda b, be: (be[0, b], 0, 0)),
        ],
        out_specs=pl.BlockSpec((BLK, H), lambda b, be: (b, 0)),
    ),
    out_shape=jax.ShapeDtypeStruct((NPAD, H), jnp.float32),
    compiler_params=pltpu.CompilerParams(vmem_limit_bytes=100 * 1024 * 1024),
)


def _combine_body(y0_ref, y1_ref, w0_ref, w1_ref, o_ref):
    w0 = w0_ref[...].reshape(BLK, 1)
    w1 = w1_ref[...].reshape(BLK, 1)
    o_ref[...] = y0_ref[...] * w0 + y1_ref[...] * w1


_combine = pl.pallas_call(
    _combine_body,
    grid=(M // BLK,),
    in_specs=[
        pl.BlockSpec((BLK, H), lambda b: (b, 0)),
        pl.BlockSpec((BLK, H), lambda b: (b, 0)),
        pl.BlockSpec((BLK,), lambda b: (b,)),
        pl.BlockSpec((BLK,), lambda b: (b,)),
    ],
    out_specs=pl.BlockSpec((BLK, H), lambda b: (b, 0)),
    out_shape=jax.ShapeDtypeStruct((M, H), jnp.float32),
)


def kernel(x, router_w, w1, w3, w2):
    bs, seqlen, dim = x.shape
    xt = x.reshape(M, H)
    pos0, pos1, wt0, wt1, be = _router(xt, router_w)
    dispatch, gather = _sc_kernels()
    xs = dispatch(xt, pos0, pos1)
    ys = _gemm(be, xs, w1, w3, w2)
    y0, y1 = gather(ys, pos0, pos1)
    out = _combine(y0, y1, wt0, wt1)
    return out.reshape(bs, seqlen, dim)


# speedup vs baseline: 1.0737x; 1.0032x over previous
import functools

import jax
import jax.numpy as jnp
from jax import lax
from jax.experimental import pallas as pl
from jax.experimental.pallas import tpu as pltpu
from jax.experimental.pallas import tpu_sc as plsc

E = 8
K = 2
H = 1024
F = 768
M = 2048
BLK = 128
NBLK = 39
NPAD = NBLK * BLK
NC, NS = 2, 16
NW = NC * NS
TPW = M // NW


def _router_body(x_ref, rw_ref, pos0_ref, pos1_ref, w0_ref, w1_ref, be_ref,
                 offb_ref, lenb_ref):
    x = x_ref[...]
    rw = rw_ref[...]
    lt = lax.dot_general(rw, x, (((0,), (1,)), ((), ())),
                         preferred_element_type=jnp.float32)

    m1 = jnp.max(lt, axis=0, keepdims=True)
    sel1 = lt == m1
    masked = jnp.where(sel1, -1e30, lt)
    m2 = jnp.max(masked, axis=0, keepdims=True)
    sel2 = masked == m2

    r = jnp.exp(m2 - m1)
    w0 = 1.0 / (1.0 + r)
    w0_ref[...] = w0.reshape(M)
    w1_ref[...] = (1.0 - w0).reshape(M)

    memb = jnp.where(sel1 | sel2, 1.0, 0.0)
    zc = jnp.zeros((E, M), jnp.float32)
    c = memb
    for k in range(11):
        s = 1 << k
        c = c + jnp.concatenate([zc[:, :s], c[:, : M - s]], axis=1)
    cex = c - memb

    counts = c[:, M - 1 : M]
    cnt_i = counts.astype(jnp.int32)
    padded_i = ((cnt_i + (BLK - 1)) >> 7) << 7
    padded_f = padded_i.astype(jnp.float32)
    zo = jnp.zeros((E, 1), jnp.float32)
    o = padded_f
    for k in range(3):
        s = 1 << k
        o = o + jnp.concatenate([zo[:s, :] * 0.0, o[: E - s, :]], axis=0)
    off = o - padded_f

    dest = off + cex
    pos0 = jnp.sum(jnp.where(sel1, dest, 0.0), axis=0)
    pos1 = jnp.sum(jnp.where(sel2, dest, 0.0), axis=0)
    pos0_ref[...] = pos0.astype(jnp.int32)
    pos1_ref[...] = pos1.astype(jnp.int32)

    ends = (off + padded_f) * (1.0 / BLK)
    bl = lax.broadcasted_iota(jnp.int32, (E, 128), 1).astype(jnp.float32)
    be = jnp.sum(jnp.where(ends <= bl, 1, 0), axis=0, keepdims=True)
    be_ref[...] = jnp.minimum(be, E - 1).astype(jnp.int32)

    offb_ref[...] = (off * (1.0 / BLK)).astype(jnp.int32)
    lenb_ref[...] = (padded_f * (1.0 / BLK)).astype(jnp.int32)


_router = pl.pallas_call(
    _router_body,
    out_shape=[
        jax.ShapeDtypeStruct((M,), jnp.int32),
        jax.ShapeDtypeStruct((M,), jnp.int32),
        jax.ShapeDtypeStruct((M,), jnp.float32),
        jax.ShapeDtypeStruct((M,), jnp.float32),
        jax.ShapeDtypeStruct((1, 128), jnp.int32),
        jax.ShapeDtypeStruct((E, 1), jnp.int32),
        jax.ShapeDtypeStruct((E, 1), jnp.int32),
    ],
    compiler_params=pltpu.CompilerParams(vmem_limit_bytes=100 * 1024 * 1024),
)


@functools.cache
def _sc_kernels():
    mesh = plsc.VectorSubcoreMesh(core_axis_name="c", subcore_axis_name="s",
                                  num_cores=NC, num_subcores=NS)

    @functools.partial(
        pl.kernel,
        out_type=jax.ShapeDtypeStruct((NPAD, H), jnp.float32),
        mesh=mesh,
        scratch_types=[
            pltpu.VMEM((TPW,), jnp.int32),
            pltpu.VMEM((TPW,), jnp.int32),
            pltpu.VMEM((TPW, H), jnp.float32),
            pltpu.SemaphoreType.DMA,
        ],
    )
    def dispatch(x_hbm, pos0_hbm, pos1_hbm, xs_hbm, idx0_v, idx1_v, rows_v, sem):
        wid = lax.axis_index("s") * NC + lax.axis_index("c")
        base = wid * TPW
        pltpu.sync_copy(pos0_hbm.at[pl.ds(base, TPW)], idx0_v)
        pltpu.sync_copy(pos1_hbm.at[pl.ds(base, TPW)], idx1_v)
        pltpu.sync_copy(x_hbm.at[pl.ds(base, TPW)], rows_v)
        a = pltpu.async_copy(rows_v, xs_hbm.at[idx0_v], sem)
        b = pltpu.async_copy(rows_v, xs_hbm.at[idx1_v], sem)
        a.wait()
        b.wait()

    @functools.partial(
        pl.kernel,
        out_type=[
            jax.ShapeDtypeStruct((M, H), jnp.float32),
            jax.ShapeDtypeStruct((M, H), jnp.float32),
        ],
        mesh=mesh,
        scratch_types=[
            pltpu.VMEM((TPW,), jnp.int32),
            pltpu.VMEM((TPW,), jnp.int32),
            pltpu.VMEM((TPW // 2, H), jnp.float32),
            pltpu.VMEM((TPW // 2, H), jnp.float32),
            pltpu.SemaphoreType.DMA,
            pltpu.SemaphoreType.DMA,
            pltpu.SemaphoreType.DMA,
            pltpu.SemaphoreType.DMA,
        ],
    )
    def gather(ys_hbm, pos0_hbm, pos1_hbm, y0_hbm, y1_hbm, idx0_v, idx1_v,
               rows_a, rows_b, sga, sgb, ssa, ssb):
        wid = lax.axis_index("s") * NC + lax.axis_index("c")
        base = wid * TPW
        Ch = TPW // 2
        pltpu.sync_copy(pos0_hbm.at[pl.ds(base, TPW)], idx0_v)
        pltpu.sync_copy(pos1_hbm.at[pl.ds(base, TPW)], idx1_v)
        g = pltpu.async_copy(ys_hbm.at[idx0_v.at[pl.ds(0, Ch)]], rows_a, sga)
        h = pltpu.async_copy(ys_hbm.at[idx0_v.at[pl.ds(Ch, Ch)]], rows_b, sgb)
        g.wait()
        s0 = pltpu.async_copy(rows_a, y0_hbm.at[pl.ds(base, Ch)], ssa)
        h.wait()
        s1 = pltpu.async_copy(rows_b, y0_hbm.at[pl.ds(base + Ch, Ch)], ssb)
        s0.wait()
        g = pltpu.async_copy(ys_hbm.at[idx1_v.at[pl.ds(0, Ch)]], rows_a, sga)
        s1.wait()
        h = pltpu.async_copy(ys_hbm.at[idx1_v.at[pl.ds(Ch, Ch)]], rows_b, sgb)
        g.wait()
        s0 = pltpu.async_copy(rows_a, y1_hbm.at[pl.ds(base, Ch)], ssa)
        h.wait()
        s1 = pltpu.async_copy(rows_b, y1_hbm.at[pl.ds(base + Ch, Ch)], ssb)
        s0.wait()
        s1.wait()

    return dispatch, gather


def _gemm_body(offb_ref, lenb_ref, xs_hbm, w1_hbm, w3_hbm, w2_hbm, out_hbm,
               w1b, w3b, w2b, xsb, outb, wsem, xsem, osem):
    def w_copies(e, slot):
        return (
            pltpu.make_async_copy(w1_hbm.at[e], w1b.at[slot], wsem.at[slot]),
            pltpu.make_async_copy(w3_hbm.at[e], w3b.at[slot], wsem.at[slot]),
            pltpu.make_async_copy(w2_hbm.at[e], w2b.at[slot], wsem.at[slot]),
        )

    def xs_copy(b, slot):
        return pltpu.make_async_copy(xs_hbm.at[pl.ds(b * BLK, BLK)],
                                     xsb.at[slot], xsem.at[slot])

    def out_copy(b, slot):
        return pltpu.make_async_copy(outb.at[slot],
                                     out_hbm.at[pl.ds(b * BLK, BLK)],
                                     osem.at[slot])

    tot = offb_ref[E - 1, 0] + lenb_ref[E - 1, 0]
    for c in w_copies(0, 0):
        c.start()
    xs_copy(0, 0).start()
    for k in range(E):
        slot = k & 1
        for c in w_copies(k, slot):
            c.wait()
        if k + 1 < E:
            for c in w_copies(k + 1, 1 - slot):
                c.start()

        def body(b, carry, kslot=slot):
            bs = b & 1
            xs_copy(b, bs).wait()

            @pl.when(b + 1 < tot)
            def _():
                xs_copy(b + 1, 1 - bs).start()

            @pl.when(b >= 2)
            def _():
                out_copy(b - 2, bs).wait()

            xb = xsb[bs]
            hh = jnp.dot(xb, w1b[kslot], preferred_element_type=jnp.float32)
            uu = jnp.dot(xb, w3b[kslot], preferred_element_type=jnp.float32)
            act = hh * (1.0 / (1.0 + jnp.exp(-hh))) * uu
            outb[bs] = jnp.dot(act, w2b[kslot], preferred_element_type=jnp.float32)
            out_copy(b, bs).start()
            return carry

        lo = offb_ref[k, 0]
        lax.fori_loop(lo, lo + lenb_ref[k, 0], body, 0)
    out_copy(tot - 2, (tot - 2) & 1).wait()
    out_copy(tot - 1, (tot - 1) & 1).wait()


_gemm = pl.pallas_call(
    _gemm_body,
    in_specs=[
        pl.BlockSpec(memory_space=pltpu.MemorySpace.SMEM),
        pl.BlockSpec(memory_space=pltpu.MemorySpace.SMEM),
        pl.BlockSpec(memory_space=pltpu.MemorySpace.HBM),
        pl.BlockSpec(memory_space=pltpu.MemorySpace.HBM),
        pl.BlockSpec(memory_space=pltpu.MemorySpace.HBM),
        pl.BlockSpec(memory_space=pltpu.MemorySpace.HBM),
    ],
    out_specs=pl.BlockSpec(memory_space=pltpu.MemorySpace.HBM),
    out_shape=jax.ShapeDtypeStruct((NPAD, H), jnp.float32),
    scratch_shapes=[
        pltpu.VMEM((2, H, F), jnp.float32),
        pltpu.VMEM((2, H, F), jnp.float32),
        pltpu.VMEM((2, F, H), jnp.float32),
        pltpu.VMEM((2, BLK, H), jnp.float32),
        pltpu.VMEM((2, BLK, H), jnp.float32),
        pltpu.SemaphoreType.DMA((2,)),
        pltpu.SemaphoreType.DMA((2,)),
        pltpu.SemaphoreType.DMA((2,)),
    ],
    compiler_params=pltpu.CompilerParams(vmem_limit_bytes=100 * 1024 * 1024),
)


def _combine_body(y0_ref, y1_ref, w0_ref, w1_ref, o_ref):
    w0 = w0_ref[...].reshape(BLK, 1)
    w1 = w1_ref[...].reshape(BLK, 1)
    o_ref[...] = y0_ref[...] * w0 + y1_ref[...] * w1


_combine = pl.pallas_call(
    _combine_body,
    grid=(M // BLK,),
    in_specs=[
        pl.BlockSpec((BLK, H), lambda b: (b, 0)),
        pl.BlockSpec((BLK, H), lambda b: (b, 0)),
        pl.BlockSpec((BLK,), lambda b: (b,)),
        pl.BlockSpec((BLK,), lambda b: (b,)),
    ],
    out_specs=pl.BlockSpec((BLK, H), lambda b: (b, 0)),
    out_shape=jax.ShapeDtypeStruct((M, H), jnp.float32),
)


def kernel(x, router_w, w1, w3, w2):
    bs, seqlen, dim = x.shape
    xt = x.reshape(M, H)
    pos0, pos1, wt0, wt1, be, offb, lenb = _router(xt, router_w)
    dispatch, gather = _sc_kernels()
    xs = dispatch(xt, pos0, pos1)
    ys = _gemm(offb, lenb, xs, w1, w3, w2)
    y0, y1 = gather(ys, pos0, pos1)
    out = _combine(y0, y1, wt0, wt1)
    return out.reshape(bs, seqlen, dim)
